# baseline (speedup 1.0000x reference)
import jax
import jax.numpy as jnp
from jax import lax
from jax.experimental import pallas as pl
from jax.experimental.pallas import tpu as pltpu
from jax.experimental.pallas import tpu_sc as plsc

EPS = 1e-5

NC = 2
NS = 16
NW = NC * NS
LANES = 16

K_EDGES = 5120
B_ROWS = 64
N_RING = 4


def _make_segmax(n_pad, n_per_tile, d, e_pad, init_val):
  dc = d // LANES
  n_chunks = e_pad // K_EDGES
  assert n_chunks % 2 == 0
  mesh = plsc.VectorSubcoreMesh(core_axis_name="c", subcore_axis_name="s")

  def body(feat_hbm, edges_hbm, out_hbm,
           ebuf0, ebuf1, csrc0, cdst0, csrc1, cdst1,
           *rest):
    rows_ring = rest[:N_RING]
    sem_e0, sem_e1 = rest[N_RING:N_RING + 2]
    sem_ring = rest[N_RING + 2:2 * N_RING + 2]
    acc_v = rest[2 * N_RING + 2]
    wid = lax.axis_index("s") * NC + lax.axis_index("c")
    lo = wid * n_per_tile

    fill = jnp.full((LANES,), init_val, jnp.float32)

    def init_row(i, _):
      for j in range(dc):
        acc_v[i, pl.ds(j * LANES, LANES)] = fill
      return 0
    lax.fori_loop(0, n_per_tile + 1, init_row, 0)

    ones16 = jnp.ones((LANES,), jnp.bool_)
    zero16 = jnp.zeros((LANES,), jnp.int32)
    scrap16 = jnp.full((LANES,), n_per_tile, jnp.int32)
    lane_iota = lax.iota(jnp.int32, LANES)
    pad16 = zero16 + wid * 8 + (lane_iota & 7)

    def issue_edges(c, ebuf, sem):
      pltpu.async_copy(edges_hbm.at[:, pl.ds(c * K_EDGES, K_EDGES)],
                       ebuf, sem)

    def wait_edges(ebuf, sem):
      pltpu.make_async_copy(edges_hbm.at[:, pl.ds(0, K_EDGES)],
                            ebuf, sem).wait()

    lov = jnp.full((LANES,), 0, jnp.int32) + lo

    def filt_chunk(ebuf, csrc, cdst):
      def filt(i, cnt):
        dvec = ebuf[1, pl.ds(i * LANES, LANES)]
        m = (dvec >= lov) & (dvec < lov + n_per_tile)
        nm = plsc.all_reduce_population_count(m)[0]

        @pl.when(nm > 0)
        def _compact():
          svec = ebuf[0, pl.ds(i * LANES, LANES)]
          mi = m.astype(jnp.int32)
          incl = plsc.cumsum(mi)
          pos = incl - mi + cnt
          plsc.store_scatter(csrc, [pos], svec, mask=m)
          plsc.store_scatter(cdst, [pos], dvec - lo, mask=m)
        return cnt + nm
      cnt = lax.fori_loop(0, K_EDGES // LANES, filt, jnp.int32(0))

      for t in range(B_ROWS // LANES):
        pos = cnt + t * LANES + lane_iota
        plsc.store_scatter(csrc, [pos], pad16, mask=ones16)
        plsc.store_scatter(cdst, [pos], scrap16, mask=ones16)
      return cnt

    def fire(csrc, b, buf, sem):
      idx = csrc.at[pl.ds(b * B_ROWS, B_ROWS)]
      pltpu.async_copy(feat_hbm.at[idx], buf, sem)

    def fire_first(csrc, cnt):
      nb = (cnt + B_ROWS - 1) // B_ROWS
      for s in range(N_RING):
        @pl.when(nb >= s + 1)
        def _g(s=s):
          fire(csrc, s, rows_ring[s], sem_ring[s])

    def rmw_batch(rows_v, cdst, base):
      def rmw_group(g, _):
        offs = cdst[pl.ds(base + g * LANES, LANES)]
        for r in range(LANES):
          off = offs[r]
          for j in range(dc):
            acc_v[off, pl.ds(j * LANES, LANES)] = jnp.maximum(
                acc_v[off, pl.ds(j * LANES, LANES)],
                rows_v[g * LANES + r, pl.ds(j * LANES, LANES)])
        return 0
      lax.fori_loop(0, B_ROWS // LANES, rmw_group, 0)

    def drain_chunk(csrc, cdst, cnt):
      nb = (cnt + B_ROWS - 1) // B_ROWS

      def drain(b, _):
        def go(cur, sem_cur):
          pltpu.make_async_copy(
              feat_hbm.at[csrc.at[pl.ds(0, B_ROWS)]], cur, sem_cur).wait()
          rmw_batch(cur, cdst, b * B_ROWS)

          @pl.when(b + N_RING < nb)
          def _refire():
            fire(csrc, b + N_RING, cur, sem_cur)

        for s in range(N_RING):
          @pl.when(b % N_RING == s)
          def _slot(s=s):
            go(rows_ring[s], sem_ring[s])
        return 0
      lax.fori_loop(0, nb, drain, 0)

    issue_edges(0, ebuf0, sem_e0)
    issue_edges(1, ebuf1, sem_e1)
    wait_edges(ebuf0, sem_e0)
    cnt0 = filt_chunk(ebuf0, csrc0, cdst0)
    fire_first(csrc0, cnt0)

    def pair_body(i, cnt_even):
      c0 = 2 * i

      @pl.when(c0 + 2 < n_chunks)
      def _issue2():
        issue_edges(c0 + 2, ebuf0, sem_e0)

      wait_edges(ebuf1, sem_e1)
      cnt1 = filt_chunk(ebuf1, csrc1, cdst1)
      drain_chunk(csrc0, cdst0, cnt_even)
      fire_first(csrc1, cnt1)

      @pl.when(c0 + 3 < n_chunks)
      def _issue3():
        issue_edges(c0 + 3, ebuf1, sem_e1)

      @pl.when(c0 + 2 < n_chunks)
      def _wait2():
        wait_edges(ebuf0, sem_e0)
      cnt2 = filt_chunk(ebuf0, csrc0, cdst0)
      drain_chunk(csrc1, cdst1, cnt1)

      @pl.when(c0 + 2 < n_chunks)
      def _fire2():
        fire_first(csrc0, cnt2)
      return cnt2

    lax.fori_loop(0, n_chunks // 2, pair_body, cnt0)

    pltpu.sync_copy(acc_v.at[pl.ds(0, n_per_tile)],
                    out_hbm.at[pl.ds(lo, n_per_tile)])

  return pl.kernel(
      body,
      out_type=jax.ShapeDtypeStruct((n_pad, d), jnp.float32),
      mesh=mesh,
      compiler_params=pltpu.CompilerParams(needs_layout_passes=False),
      scratch_types=[
          pltpu.VMEM((2, K_EDGES), jnp.int32),
          pltpu.VMEM((2, K_EDGES), jnp.int32),
          pltpu.VMEM((K_EDGES + B_ROWS,), jnp.int32),
          pltpu.VMEM((K_EDGES + B_ROWS,), jnp.int32),
          pltpu.VMEM((K_EDGES + B_ROWS,), jnp.int32),
          pltpu.VMEM((K_EDGES + B_ROWS,), jnp.int32),
          *[pltpu.VMEM((B_ROWS, d), jnp.float32) for _ in range(N_RING)],
          pltpu.SemaphoreType.DMA,
          pltpu.SemaphoreType.DMA,
          *[pltpu.SemaphoreType.DMA for _ in range(N_RING)],
          pltpu.VMEM((n_per_tile + 1, d), jnp.float32),
      ],
  )


def _layernorm(h, g, b):
  mu = jnp.mean(h, axis=-1, keepdims=True)
  var = jnp.mean((h - mu) ** 2, axis=-1, keepdims=True)
  return (h - mu) / jnp.sqrt(var + EPS) * g + b


def _tc1_body(agg_ref, x_ref, wl_ref, b_ref, wr_ref, g_ref, be_ref, o_ref):
  a = agg_ref[...]
  a = jnp.where(a == -jnp.inf, 0.0, a)
  h = (jnp.dot(a, wl_ref[...], preferred_element_type=jnp.float32)
       + jnp.dot(x_ref[...], wr_ref[...], preferred_element_type=jnp.float32)
       + b_ref[...])
  h = _layernorm(h, g_ref[...], be_ref[...])
  h = jnp.maximum(h, 0.0)
  o_ref[...] = jnp.concatenate([h, jnp.zeros_like(h)], axis=1)


def _tc2_body(agg_ref, h1_ref, wl_ref, b_ref, wr_ref, g_ref, be_ref,
              wm1_ref, bm1_ref, wm2_ref, bm2_ref, o_ref):
  d_h = wl_ref.shape[0]
  agg = agg_ref[...][:, :d_h]
  h1 = h1_ref[...][:, :d_h]
  h = (jnp.dot(agg, wl_ref[...], preferred_element_type=jnp.float32)
       + jnp.dot(h1, wr_ref[...], preferred_element_type=jnp.float32)
       + b_ref[...])
  h = _layernorm(h, g_ref[...], be_ref[...])
  h = jnp.maximum(h, 0.0)
  z = jnp.maximum(
      jnp.dot(h, wm1_ref[...], preferred_element_type=jnp.float32)
      + bm1_ref[...], 0.0)
  y = jnp.dot(z, wm2_ref[...], preferred_element_type=jnp.float32) + bm2_ref[...]
  o_ref[...] = jax.nn.sigmoid(y).reshape(o_ref.shape)


def _const_spec(shape):
  return pl.BlockSpec(shape, lambda i: (0,) * len(shape))


def kernel(x, edge_index, W_l1, b_l1, W_r1, W_l2, b_l2, W_r2,
           g1, be1, g2, be2, Wm1, bm1, Wm2, bm2):
  n, d_in = x.shape
  d_h = W_l1.shape[0]
  e = edge_index.shape[1]

  n_per_tile = ((n + NW - 1) // NW + 7) // 8 * 8
  n_pad = n_per_tile * NW
  e_pad = ((e + 2 * K_EDGES - 1) // (2 * K_EDGES)) * (2 * K_EDGES)

  src = edge_index[0].astype(jnp.int32)
  dst = edge_index[1].astype(jnp.int32)
  if e_pad != e:
    src = jnp.pad(src, (0, e_pad - e))
    dst = jnp.pad(dst, (0, e_pad - e), constant_values=n_pad)
  edges = jnp.stack([src, dst])
  x_pad = jnp.pad(x, ((0, n_pad - n), (0, 0)))

  segmax1 = _make_segmax(n_pad, n_per_tile, d_in, e_pad, -jnp.inf)
  segmax2 = _make_segmax(n_pad, n_per_tile, d_in, e_pad, 0.0)

  agg1 = segmax1(x_pad, edges)

  blk = 1024
  grid = (n_pad // blk,)
  row_spec = lambda dd: pl.BlockSpec((blk, dd), lambda i: (i, 0))

  h1 = pl.pallas_call(
      _tc1_body,
      grid=grid,
      in_specs=[row_spec(d_in), row_spec(d_in),
                _const_spec((d_in, d_h)), _const_spec((1, d_h)),
                _const_spec((d_in, d_h)), _const_spec((1, d_h)),
                _const_spec((1, d_h))],
      out_specs=row_spec(2 * d_h),
      out_shape=jax.ShapeDtypeStruct((n_pad, 2 * d_h), jnp.float32),
  )(agg1, x_pad, W_l1.T, b_l1.reshape(1, -1), W_r1.T,
    g1.reshape(1, -1), be1.reshape(1, -1))

  agg2 = segmax2(h1, edges)

  d_m = Wm1.shape[0]
  out = pl.pallas_call(
      _tc2_body,
      grid=grid,
      in_specs=[row_spec(2 * d_h), row_spec(2 * d_h),
                _const_spec((d_h, d_h)), _const_spec((1, d_h)),
                _const_spec((d_h, d_h)), _const_spec((1, d_h)),
                _const_spec((1, d_h)),
                _const_spec((d_h, d_m)), _const_spec((1, d_m)),
                _const_spec((d_m, 1)), _const_spec((1, 1))],
      out_specs=pl.BlockSpec((blk // 128, 128), lambda i: (i, 0)),
      out_shape=jax.ShapeDtypeStruct((n_pad // 128, 128), jnp.float32),
  )(agg2, h1, W_l2.T, b_l2.reshape(1, -1), W_r2.T,
    g2.reshape(1, -1), be2.reshape(1, -1),
    Wm1.T, bm1.reshape(1, -1), Wm2.T, bm2.reshape(1, -1))

  return out.reshape(-1)[:n]


# speedup vs baseline: 1.4279x; 1.4279x over previous
import jax
import jax.numpy as jnp
from jax import lax
from jax.experimental import pallas as pl
from jax.experimental.pallas import tpu as pltpu
from jax.experimental.pallas import tpu_sc as plsc

EPS = 1e-5

NC = 2
NS = 16
NW = NC * NS
LANES = 16

K_EDGES = 5120
B_ROWS = 64
N_RING = 4


def _make_segmax(n_pad, n_per_tile, d, e_pad, init_val):
  dc = d // LANES
  n_chunks = e_pad // K_EDGES
  assert n_chunks % 2 == 0
  mesh = plsc.VectorSubcoreMesh(core_axis_name="c", subcore_axis_name="s")

  def body(feat_hbm, edges_hbm, out_hbm,
           ebuf0, ebuf1, csrc0, cdst0, csrc1, cdst1,
           *rest):
    rows_ring = rest[:N_RING]
    sem_e0, sem_e1 = rest[N_RING:N_RING + 2]
    sem_ring = rest[N_RING + 2:2 * N_RING + 2]
    acc_v = rest[2 * N_RING + 2]
    wid = lax.axis_index("s") * NC + lax.axis_index("c")
    lo = wid * n_per_tile

    fill = jnp.full((LANES,), init_val, jnp.float32)

    def init_row(i, _):
      for j in range(dc):
        acc_v[i, pl.ds(j * LANES, LANES)] = fill
      return 0
    lax.fori_loop(0, n_per_tile + 1, init_row, 0)

    ones16 = jnp.ones((LANES,), jnp.bool_)
    zero16 = jnp.zeros((LANES,), jnp.int32)
    scrap16 = jnp.full((LANES,), n_per_tile, jnp.int32)
    lane_iota = lax.iota(jnp.int32, LANES)
    pad16 = zero16 + wid * 8 + (lane_iota & 7)

    def issue_edges(c, ebuf, sem):
      pltpu.async_copy(edges_hbm.at[:, pl.ds(c * K_EDGES, K_EDGES)],
                       ebuf, sem)

    def wait_edges(ebuf, sem):
      pltpu.make_async_copy(edges_hbm.at[:, pl.ds(0, K_EDGES)],
                            ebuf, sem).wait()

    lov = jnp.full((LANES,), 0, jnp.int32) + lo

    def filt_chunk(ebuf, csrc, cdst):
      def filt(i, cnt):
        sub = []
        for u in range(2):
          off = (2 * i + u) * LANES
          svec = ebuf[0, pl.ds(off, LANES)]
          dvec = ebuf[1, pl.ds(off, LANES)]
          m = (dvec >= lov) & (dvec < lov + n_per_tile)
          mi = m.astype(jnp.int32)
          incl = plsc.cumsum(mi)
          sub.append((svec, dvec, m, mi, incl))
        for svec, dvec, m, mi, incl in sub:
          pos = incl - mi + cnt
          plsc.store_scatter(csrc, [pos], svec, mask=m)
          plsc.store_scatter(cdst, [pos], dvec - lo, mask=m)
          cnt = cnt + incl[LANES - 1]
        return cnt
      cnt = lax.fori_loop(0, K_EDGES // (2 * LANES), filt, jnp.int32(0))

      for t in range(B_ROWS // LANES):
        pos = cnt + t * LANES + lane_iota
        plsc.store_scatter(csrc, [pos], pad16, mask=ones16)
        plsc.store_scatter(cdst, [pos], scrap16, mask=ones16)
      return cnt

    def fire(csrc, b, buf, sem):
      idx = csrc.at[pl.ds(b * B_ROWS, B_ROWS)]
      pltpu.async_copy(feat_hbm.at[idx], buf, sem)

    def fire_first(csrc, cnt):
      nb = (cnt + B_ROWS - 1) // B_ROWS
      for s in range(N_RING):
        @pl.when(nb >= s + 1)
        def _g(s=s):
          fire(csrc, s, rows_ring[s], sem_ring[s])

    def rmw_batch(rows_v, cdst, base):
      def rmw_group(g, _):
        offs = cdst[pl.ds(base + g * LANES, LANES)]
        for r in range(LANES):
          off = offs[r]
          for j in range(dc):
            acc_v[off, pl.ds(j * LANES, LANES)] = jnp.maximum(
                acc_v[off, pl.ds(j * LANES, LANES)],
                rows_v[g * LANES + r, pl.ds(j * LANES, LANES)])
        return 0
      lax.fori_loop(0, B_ROWS // LANES, rmw_group, 0)

    def drain_chunk(csrc, cdst, cnt):
      nb = (cnt + B_ROWS - 1) // B_ROWS

      def drain(b, _):
        def go(cur, sem_cur):
          pltpu.make_async_copy(
              feat_hbm.at[csrc.at[pl.ds(0, B_ROWS)]], cur, sem_cur).wait()
          rmw_batch(cur, cdst, b * B_ROWS)

          @pl.when(b + N_RING < nb)
          def _refire():
            fire(csrc, b + N_RING, cur, sem_cur)

        for s in range(N_RING):
          @pl.when(b % N_RING == s)
          def _slot(s=s):
            go(rows_ring[s], sem_ring[s])
        return 0
      lax.fori_loop(0, nb, drain, 0)

    issue_edges(0, ebuf0, sem_e0)
    issue_edges(1, ebuf1, sem_e1)
    wait_edges(ebuf0, sem_e0)
    cnt0 = filt_chunk(ebuf0, csrc0, cdst0)
    fire_first(csrc0, cnt0)

    def pair_body(i, cnt_even):
      c0 = 2 * i

      @pl.when(c0 + 2 < n_chunks)
      def _issue2():
        issue_edges(c0 + 2, ebuf0, sem_e0)

      wait_edges(ebuf1, sem_e1)
      cnt1 = filt_chunk(ebuf1, csrc1, cdst1)
      drain_chunk(csrc0, cdst0, cnt_even)
      fire_first(csrc1, cnt1)

      @pl.when(c0 + 3 < n_chunks)
      def _issue3():
        issue_edges(c0 + 3, ebuf1, sem_e1)

      @pl.when(c0 + 2 < n_chunks)
      def _wait2():
        wait_edges(ebuf0, sem_e0)
      cnt2 = filt_chunk(ebuf0, csrc0, cdst0)
      drain_chunk(csrc1, cdst1, cnt1)

      @pl.when(c0 + 2 < n_chunks)
      def _fire2():
        fire_first(csrc0, cnt2)
      return cnt2

    lax.fori_loop(0, n_chunks // 2, pair_body, cnt0)

    pltpu.sync_copy(acc_v.at[pl.ds(0, n_per_tile)],
                    out_hbm.at[pl.ds(lo, n_per_tile)])

  return pl.kernel(
      body,
      out_type=jax.ShapeDtypeStruct((n_pad, d), jnp.float32),
      mesh=mesh,
      compiler_params=pltpu.CompilerParams(needs_layout_passes=False),
      scratch_types=[
          pltpu.VMEM((2, K_EDGES), jnp.int32),
          pltpu.VMEM((2, K_EDGES), jnp.int32),
          pltpu.VMEM((K_EDGES + B_ROWS,), jnp.int32),
          pltpu.VMEM((K_EDGES + B_ROWS,), jnp.int32),
          pltpu.VMEM((K_EDGES + B_ROWS,), jnp.int32),
          pltpu.VMEM((K_EDGES + B_ROWS,), jnp.int32),
          *[pltpu.VMEM((B_ROWS, d), jnp.float32) for _ in range(N_RING)],
          pltpu.SemaphoreType.DMA,
          pltpu.SemaphoreType.DMA,
          *[pltpu.SemaphoreType.DMA for _ in range(N_RING)],
          pltpu.VMEM((n_per_tile + 1, d), jnp.float32),
      ],
  )


def _layernorm(h, g, b):
  mu = jnp.mean(h, axis=-1, keepdims=True)
  var = jnp.mean((h - mu) ** 2, axis=-1, keepdims=True)
  return (h - mu) / jnp.sqrt(var + EPS) * g + b


def _tc1_body(agg_ref, x_ref, wl_ref, b_ref, wr_ref, g_ref, be_ref, o_ref):
  a = agg_ref[...]
  a = jnp.where(a == -jnp.inf, 0.0, a)
  h = (jnp.dot(a, wl_ref[...], preferred_element_type=jnp.float32)
       + jnp.dot(x_ref[...], wr_ref[...], preferred_element_type=jnp.float32)
       + b_ref[...])
  h = _layernorm(h, g_ref[...], be_ref[...])
  h = jnp.maximum(h, 0.0)
  o_ref[...] = jnp.concatenate([h, jnp.zeros_like(h)], axis=1)


def _tc2_body(agg_ref, h1_ref, wl_ref, b_ref, wr_ref, g_ref, be_ref,
              wm1_ref, bm1_ref, wm2_ref, bm2_ref, o_ref):
  d_h = wl_ref.shape[0]
  agg = agg_ref[...][:, :d_h]
  h1 = h1_ref[...][:, :d_h]
  h = (jnp.dot(agg, wl_ref[...], preferred_element_type=jnp.float32)
       + jnp.dot(h1, wr_ref[...], preferred_element_type=jnp.float32)
       + b_ref[...])
  h = _layernorm(h, g_ref[...], be_ref[...])
  h = jnp.maximum(h, 0.0)
  z = jnp.maximum(
      jnp.dot(h, wm1_ref[...], preferred_element_type=jnp.float32)
      + bm1_ref[...], 0.0)
  y = jnp.dot(z, wm2_ref[...], preferred_element_type=jnp.float32) + bm2_ref[...]
  o_ref[...] = jax.nn.sigmoid(y).reshape(o_ref.shape)


def _const_spec(shape):
  return pl.BlockSpec(shape, lambda i: (0,) * len(shape))


def kernel(x, edge_index, W_l1, b_l1, W_r1, W_l2, b_l2, W_r2,
           g1, be1, g2, be2, Wm1, bm1, Wm2, bm2):
  n, d_in = x.shape
  d_h = W_l1.shape[0]
  e = edge_index.shape[1]

  n_per_tile = ((n + NW - 1) // NW + 7) // 8 * 8
  n_pad = n_per_tile * NW
  e_pad = ((e + 2 * K_EDGES - 1) // (2 * K_EDGES)) * (2 * K_EDGES)

  src = edge_index[0].astype(jnp.int32)
  dst = edge_index[1].astype(jnp.int32)
  if e_pad != e:
    src = jnp.pad(src, (0, e_pad - e))
    dst = jnp.pad(dst, (0, e_pad - e), constant_values=n_pad)
  edges = jnp.stack([src, dst])
  x_pad = jnp.pad(x, ((0, n_pad - n), (0, 0)))

  segmax1 = _make_segmax(n_pad, n_per_tile, d_in, e_pad, -jnp.inf)
  segmax2 = _make_segmax(n_pad, n_per_tile, d_in, e_pad, 0.0)

  agg1 = segmax1(x_pad, edges)

  blk = 1024
  grid = (n_pad // blk,)
  row_spec = lambda dd: pl.BlockSpec((blk, dd), lambda i: (i, 0))

  h1 = pl.pallas_call(
      _tc1_body,
      grid=grid,
      in_specs=[row_spec(d_in), row_spec(d_in),
                _const_spec((d_in, d_h)), _const_spec((1, d_h)),
                _const_spec((d_in, d_h)), _const_spec((1, d_h)),
                _const_spec((1, d_h))],
      out_specs=row_spec(2 * d_h),
      out_shape=jax.ShapeDtypeStruct((n_pad, 2 * d_h), jnp.float32),
  )(agg1, x_pad, W_l1.T, b_l1.reshape(1, -1), W_r1.T,
    g1.reshape(1, -1), be1.reshape(1, -1))

  agg2 = segmax2(h1, edges)

  d_m = Wm1.shape[0]
  out = pl.pallas_call(
      _tc2_body,
      grid=grid,
      in_specs=[row_spec(2 * d_h), row_spec(2 * d_h),
                _const_spec((d_h, d_h)), _const_spec((1, d_h)),
                _const_spec((d_h, d_h)), _const_spec((1, d_h)),
                _const_spec((1, d_h)),
                _const_spec((d_h, d_m)), _const_spec((1, d_m)),
                _const_spec((d_m, 1)), _const_spec((1, 1))],
      out_specs=pl.BlockSpec((blk // 128, 128), lambda i: (i, 0)),
      out_shape=jax.ShapeDtypeStruct((n_pad // 128, 128), jnp.float32),
  )(agg2, h1, W_l2.T, b_l2.reshape(1, -1), W_r2.T,
    g2.reshape(1, -1), be2.reshape(1, -1),
    Wm1.T, bm1.reshape(1, -1), Wm2.T, bm2.reshape(1, -1))

  return out.reshape(-1)[:n]


# speedup vs baseline: 1.6732x; 1.1717x over previous
import jax
import jax.numpy as jnp
from jax import lax
from jax.experimental import pallas as pl
from jax.experimental.pallas import tpu as pltpu
from jax.experimental.pallas import tpu_sc as plsc

EPS = 1e-5

NC = 2
NS = 16
NW = NC * NS
LANES = 16

K_EDGES = 5120
B_ROWS = 64
N_RING = 4


def _make_segmax(n_pad, n_per_tile, d, e_pad, init_val):
  dc = d // LANES
  n_chunks = e_pad // K_EDGES
  assert n_chunks % 2 == 0
  mesh = plsc.VectorSubcoreMesh(core_axis_name="c", subcore_axis_name="s")

  def body(feat_hbm, edges_hbm, out_hbm,
           ebuf0, ebuf1, csrc0, cdst0, csrc1, cdst1,
           *rest):
    rows_ring = rest[:N_RING]
    sem_e0, sem_e1 = rest[N_RING:N_RING + 2]
    sem_ring = rest[N_RING + 2:2 * N_RING + 2]
    acc_v = rest[2 * N_RING + 2]
    wid = lax.axis_index("s") * NC + lax.axis_index("c")
    lo = wid * n_per_tile

    fill = jnp.full((LANES,), init_val, jnp.float32)

    def init_row(i, _):
      for j in range(dc):
        acc_v[i, pl.ds(j * LANES, LANES)] = fill
      return 0
    lax.fori_loop(0, n_per_tile + 1, init_row, 0)

    ones16 = jnp.ones((LANES,), jnp.bool_)
    zero16 = jnp.zeros((LANES,), jnp.int32)
    scrap16 = jnp.full((LANES,), n_per_tile, jnp.int32)
    lane_iota = lax.iota(jnp.int32, LANES)
    pad16 = zero16 + wid * 8 + (lane_iota & 7)

    def issue_edges(c, ebuf, sem):
      pltpu.async_copy(edges_hbm.at[:, pl.ds(c * K_EDGES, K_EDGES)],
                       ebuf, sem)

    def wait_edges(ebuf, sem):
      pltpu.make_async_copy(edges_hbm.at[:, pl.ds(0, K_EDGES)],
                            ebuf, sem).wait()

    lov = jnp.full((LANES,), 0, jnp.int32) + lo

    def filt_chunk(ebuf, csrc, cdst):
      def filt(i, cnt):
        sub = []
        for u in range(4):
          off = (4 * i + u) * LANES
          svec = ebuf[0, pl.ds(off, LANES)]
          dvec = ebuf[1, pl.ds(off, LANES)]
          m = (dvec >= lov) & (dvec < lov + n_per_tile)
          mi = m.astype(jnp.int32)
          incl = plsc.cumsum(mi)
          sub.append((svec, dvec, m, mi, incl))
        for svec, dvec, m, mi, incl in sub:
          pos = incl - mi + cnt
          plsc.store_scatter(csrc, [pos], svec, mask=m)
          plsc.store_scatter(cdst, [pos], dvec - lo, mask=m)
          cnt = cnt + incl[LANES - 1]
        return cnt
      cnt = lax.fori_loop(0, K_EDGES // (4 * LANES), filt, jnp.int32(0))

      for t in range(B_ROWS // LANES):
        pos = cnt + t * LANES + lane_iota
        plsc.store_scatter(csrc, [pos], pad16, mask=ones16)
        plsc.store_scatter(cdst, [pos], scrap16, mask=ones16)
      return cnt

    def fire(csrc, b, buf, sem):
      idx = csrc.at[pl.ds(b * B_ROWS, B_ROWS)]
      pltpu.async_copy(feat_hbm.at[idx], buf, sem)

    def fire_first(csrc, cnt):
      nb = (cnt + B_ROWS - 1) // B_ROWS
      for s in range(N_RING):
        @pl.when(nb >= s + 1)
        def _g(s=s):
          fire(csrc, s, rows_ring[s], sem_ring[s])

    def rmw_batch(rows_v, cdst, base):
      def rmw_group(g, _):
        offs = cdst[pl.ds(base + g * LANES, LANES)]
        for r in range(LANES):
          off = offs[r]
          for j in range(dc):
            acc_v[off, pl.ds(j * LANES, LANES)] = jnp.maximum(
                acc_v[off, pl.ds(j * LANES, LANES)],
                rows_v[g * LANES + r, pl.ds(j * LANES, LANES)])
        return 0
      lax.fori_loop(0, B_ROWS // LANES, rmw_group, 0)

    def drain_chunk(csrc, cdst, cnt):
      nb = (cnt + B_ROWS - 1) // B_ROWS

      def drain(b, _):
        def go(cur, sem_cur):
          pltpu.make_async_copy(
              feat_hbm.at[csrc.at[pl.ds(0, B_ROWS)]], cur, sem_cur).wait()
          rmw_batch(cur, cdst, b * B_ROWS)

          @pl.when(b + N_RING < nb)
          def _refire():
            fire(csrc, b + N_RING, cur, sem_cur)

        for s in range(N_RING):
          @pl.when(b % N_RING == s)
          def _slot(s=s):
            go(rows_ring[s], sem_ring[s])
        return 0
      lax.fori_loop(0, nb, drain, 0)

    issue_edges(0, ebuf0, sem_e0)
    issue_edges(1, ebuf1, sem_e1)
    wait_edges(ebuf0, sem_e0)
    cnt0 = filt_chunk(ebuf0, csrc0, cdst0)
    fire_first(csrc0, cnt0)

    def pair_body(i, cnt_even):
      c0 = 2 * i

      @pl.when(c0 + 2 < n_chunks)
      def _issue2():
        issue_edges(c0 + 2, ebuf0, sem_e0)

      wait_edges(ebuf1, sem_e1)
      cnt1 = filt_chunk(ebuf1, csrc1, cdst1)
      drain_chunk(csrc0, cdst0, cnt_even)
      fire_first(csrc1, cnt1)

      @pl.when(c0 + 3 < n_chunks)
      def _issue3():
        issue_edges(c0 + 3, ebuf1, sem_e1)

      @pl.when(c0 + 2 < n_chunks)
      def _wait2():
        wait_edges(ebuf0, sem_e0)
      cnt2 = filt_chunk(ebuf0, csrc0, cdst0)
      drain_chunk(csrc1, cdst1, cnt1)

      @pl.when(c0 + 2 < n_chunks)
      def _fire2():
        fire_first(csrc0, cnt2)
      return cnt2

    lax.fori_loop(0, n_chunks // 2, pair_body, cnt0)

    pltpu.sync_copy(acc_v.at[pl.ds(0, n_per_tile)],
                    out_hbm.at[pl.ds(lo, n_per_tile)])

  return pl.kernel(
      body,
      out_type=jax.ShapeDtypeStruct((n_pad, d), jnp.float32),
      mesh=mesh,
      compiler_params=pltpu.CompilerParams(needs_layout_passes=False),
      scratch_types=[
          pltpu.VMEM((2, K_EDGES), jnp.int32),
          pltpu.VMEM((2, K_EDGES), jnp.int32),
          pltpu.VMEM((K_EDGES + B_ROWS,), jnp.int32),
          pltpu.VMEM((K_EDGES + B_ROWS,), jnp.int32),
          pltpu.VMEM((K_EDGES + B_ROWS,), jnp.int32),
          pltpu.VMEM((K_EDGES + B_ROWS,), jnp.int32),
          *[pltpu.VMEM((B_ROWS, d), jnp.float32) for _ in range(N_RING)],
          pltpu.SemaphoreType.DMA,
          pltpu.SemaphoreType.DMA,
          *[pltpu.SemaphoreType.DMA for _ in range(N_RING)],
          pltpu.VMEM((n_per_tile + 1, d), jnp.float32),
      ],
  )


def _layernorm(h, g, b):
  mu = jnp.mean(h, axis=-1, keepdims=True)
  var = jnp.mean((h - mu) ** 2, axis=-1, keepdims=True)
  return (h - mu) / jnp.sqrt(var + EPS) * g + b


def _tc1_body(agg_ref, x_ref, wl_ref, b_ref, wr_ref, g_ref, be_ref, o_ref):
  a = agg_ref[...]
  a = jnp.where(a == -jnp.inf, 0.0, a)
  h = (jnp.dot(a, wl_ref[...], preferred_element_type=jnp.float32)
       + jnp.dot(x_ref[...], wr_ref[...], preferred_element_type=jnp.float32)
       + b_ref[...])
  h = _layernorm(h, g_ref[...], be_ref[...])
  h = jnp.maximum(h, 0.0)
  o_ref[...] = jnp.concatenate([h, jnp.zeros_like(h)], axis=1)


def _tc2_body(agg_ref, h1_ref, wl_ref, b_ref, wr_ref, g_ref, be_ref,
              wm1_ref, bm1_ref, wm2_ref, bm2_ref, o_ref):
  d_h = wl_ref.shape[0]
  agg = agg_ref[...][:, :d_h]
  h1 = h1_ref[...][:, :d_h]
  h = (jnp.dot(agg, wl_ref[...], preferred_element_type=jnp.float32)
       + jnp.dot(h1, wr_ref[...], preferred_element_type=jnp.float32)
       + b_ref[...])
  h = _layernorm(h, g_ref[...], be_ref[...])
  h = jnp.maximum(h, 0.0)
  z = jnp.maximum(
      jnp.dot(h, wm1_ref[...], preferred_element_type=jnp.float32)
      + bm1_ref[...], 0.0)
  y = jnp.dot(z, wm2_ref[...], preferred_element_type=jnp.float32) + bm2_ref[...]
  o_ref[...] = jax.nn.sigmoid(y).reshape(o_ref.shape)


def _const_spec(shape):
  return pl.BlockSpec(shape, lambda i: (0,) * len(shape))


def kernel(x, edge_index, W_l1, b_l1, W_r1, W_l2, b_l2, W_r2,
           g1, be1, g2, be2, Wm1, bm1, Wm2, bm2):
  n, d_in = x.shape
  d_h = W_l1.shape[0]
  e = edge_index.shape[1]

  n_per_tile = ((n + NW - 1) // NW + 7) // 8 * 8
  n_pad = n_per_tile * NW
  e_pad = ((e + 2 * K_EDGES - 1) // (2 * K_EDGES)) * (2 * K_EDGES)

  src = edge_index[0].astype(jnp.int32)
  dst = edge_index[1].astype(jnp.int32)
  if e_pad != e:
    src = jnp.pad(src, (0, e_pad - e))
    dst = jnp.pad(dst, (0, e_pad - e), constant_values=n_pad)
  edges = jnp.stack([src, dst])
  x_pad = jnp.pad(x, ((0, n_pad - n), (0, 0)))

  segmax1 = _make_segmax(n_pad, n_per_tile, d_in, e_pad, -jnp.inf)
  segmax2 = _make_segmax(n_pad, n_per_tile, d_in, e_pad, 0.0)

  agg1 = segmax1(x_pad, edges)

  blk = 1024
  grid = (n_pad // blk,)
  row_spec = lambda dd: pl.BlockSpec((blk, dd), lambda i: (i, 0))

  h1 = pl.pallas_call(
      _tc1_body,
      grid=grid,
      in_specs=[row_spec(d_in), row_spec(d_in),
                _const_spec((d_in, d_h)), _const_spec((1, d_h)),
                _const_spec((d_in, d_h)), _const_spec((1, d_h)),
                _const_spec((1, d_h))],
      out_specs=row_spec(2 * d_h),
      out_shape=jax.ShapeDtypeStruct((n_pad, 2 * d_h), jnp.float32),
  )(agg1, x_pad, W_l1.T, b_l1.reshape(1, -1), W_r1.T,
    g1.reshape(1, -1), be1.reshape(1, -1))

  agg2 = segmax2(h1, edges)

  d_m = Wm1.shape[0]
  out = pl.pallas_call(
      _tc2_body,
      grid=grid,
      in_specs=[row_spec(2 * d_h), row_spec(2 * d_h),
                _const_spec((d_h, d_h)), _const_spec((1, d_h)),
                _const_spec((d_h, d_h)), _const_spec((1, d_h)),
                _const_spec((1, d_h)),
                _const_spec((d_h, d_m)), _const_spec((1, d_m)),
                _const_spec((d_m, 1)), _const_spec((1, 1))],
      out_specs=pl.BlockSpec((blk // 128, 128), lambda i: (i, 0)),
      out_shape=jax.ShapeDtypeStruct((n_pad // 128, 128), jnp.float32),
  )(agg2, h1, W_l2.T, b_l2.reshape(1, -1), W_r2.T,
    g2.reshape(1, -1), be2.reshape(1, -1),
    Wm1.T, bm1.reshape(1, -1), Wm2.T, bm2.reshape(1, -1))

  return out.reshape(-1)[:n]


# speedup vs baseline: 1.7188x; 1.0273x over previous
import jax
import jax.numpy as jnp
from jax import lax
from jax.experimental import pallas as pl
from jax.experimental.pallas import tpu as pltpu
from jax.experimental.pallas import tpu_sc as plsc

EPS = 1e-5

NC = 2
NS = 16
NW = NC * NS
LANES = 16

K_EDGES = 5120
B_ROWS = 64
N_RING = 4


def _make_segmax(n_pad, n_per_tile, d, e_pad, init_val):
  dc = d // LANES
  n_chunks = e_pad // K_EDGES
  assert n_chunks % 2 == 0
  mesh = plsc.VectorSubcoreMesh(core_axis_name="c", subcore_axis_name="s")

  def body(feat_hbm, edges_hbm, out_hbm,
           ebuf0, ebuf1, csrc0, cdst0, csrc1, cdst1,
           *rest):
    rows_ring = rest[:N_RING]
    sem_e0, sem_e1 = rest[N_RING:N_RING + 2]
    sem_ring = rest[N_RING + 2:2 * N_RING + 2]
    acc_v = rest[2 * N_RING + 2]
    wid = lax.axis_index("s") * NC + lax.axis_index("c")
    lo = wid * n_per_tile

    fill = jnp.full((LANES,), init_val, jnp.float32)

    def init_row(i, _):
      for j in range(dc):
        acc_v[i, pl.ds(j * LANES, LANES)] = fill
      return 0
    lax.fori_loop(0, n_per_tile + 1, init_row, 0)

    ones16 = jnp.ones((LANES,), jnp.bool_)
    zero16 = jnp.zeros((LANES,), jnp.int32)
    scrap16 = jnp.full((LANES,), n_per_tile, jnp.int32)
    lane_iota = lax.iota(jnp.int32, LANES)
    pad16 = zero16 + wid * 8 + (lane_iota & 7)

    def issue_edges(c, ebuf, sem):
      pltpu.async_copy(edges_hbm.at[:, pl.ds(c * K_EDGES, K_EDGES)],
                       ebuf, sem)

    def wait_edges(ebuf, sem):
      pltpu.make_async_copy(edges_hbm.at[:, pl.ds(0, K_EDGES)],
                            ebuf, sem).wait()

    lov = jnp.full((LANES,), 0, jnp.int32) + lo

    def filt_chunk(ebuf, csrc, cdst):
      def filt(i, cnt):
        sub = []
        for u in range(8):
          off = (8 * i + u) * LANES
          svec = ebuf[0, pl.ds(off, LANES)]
          dvec = ebuf[1, pl.ds(off, LANES)]
          m = (dvec >= lov) & (dvec < lov + n_per_tile)
          mi = m.astype(jnp.int32)
          incl = plsc.cumsum(mi)
          sub.append((svec, dvec, m, mi, incl))
        for svec, dvec, m, mi, incl in sub:
          pos = incl - mi + cnt
          plsc.store_scatter(csrc, [pos], svec, mask=m)
          plsc.store_scatter(cdst, [pos], dvec - lo, mask=m)
          cnt = cnt + incl[LANES - 1]
        return cnt
      cnt = lax.fori_loop(0, K_EDGES // (8 * LANES), filt, jnp.int32(0))

      for t in range(B_ROWS // LANES):
        pos = cnt + t * LANES + lane_iota
        plsc.store_scatter(csrc, [pos], pad16, mask=ones16)
        plsc.store_scatter(cdst, [pos], scrap16, mask=ones16)
      return cnt

    def fire(csrc, b, buf, sem):
      idx = csrc.at[pl.ds(b * B_ROWS, B_ROWS)]
      pltpu.async_copy(feat_hbm.at[idx], buf, sem)

    def fire_first(csrc, cnt):
      nb = (cnt + B_ROWS - 1) // B_ROWS
      for s in range(N_RING):
        @pl.when(nb >= s + 1)
        def _g(s=s):
          fire(csrc, s, rows_ring[s], sem_ring[s])

    def rmw_batch(rows_v, cdst, base):
      def rmw_group(g, _):
        offs = cdst[pl.ds(base + g * LANES, LANES)]
        for r in range(LANES):
          off = offs[r]
          for j in range(dc):
            acc_v[off, pl.ds(j * LANES, LANES)] = jnp.maximum(
                acc_v[off, pl.ds(j * LANES, LANES)],
                rows_v[g * LANES + r, pl.ds(j * LANES, LANES)])
        return 0
      lax.fori_loop(0, B_ROWS // LANES, rmw_group, 0)

    def drain_chunk(csrc, cdst, cnt):
      nb = (cnt + B_ROWS - 1) // B_ROWS

      def drain(b, _):
        def go(cur, sem_cur):
          pltpu.make_async_copy(
              feat_hbm.at[csrc.at[pl.ds(0, B_ROWS)]], cur, sem_cur).wait()
          rmw_batch(cur, cdst, b * B_ROWS)

          @pl.when(b + N_RING < nb)
          def _refire():
            fire(csrc, b + N_RING, cur, sem_cur)

        for s in range(N_RING):
          @pl.when(b % N_RING == s)
          def _slot(s=s):
            go(rows_ring[s], sem_ring[s])
        return 0
      lax.fori_loop(0, nb, drain, 0)

    issue_edges(0, ebuf0, sem_e0)
    issue_edges(1, ebuf1, sem_e1)
    wait_edges(ebuf0, sem_e0)
    cnt0 = filt_chunk(ebuf0, csrc0, cdst0)
    fire_first(csrc0, cnt0)

    def pair_body(i, cnt_even):
      c0 = 2 * i

      @pl.when(c0 + 2 < n_chunks)
      def _issue2():
        issue_edges(c0 + 2, ebuf0, sem_e0)

      wait_edges(ebuf1, sem_e1)
      cnt1 = filt_chunk(ebuf1, csrc1, cdst1)
      drain_chunk(csrc0, cdst0, cnt_even)
      fire_first(csrc1, cnt1)

      @pl.when(c0 + 3 < n_chunks)
      def _issue3():
        issue_edges(c0 + 3, ebuf1, sem_e1)

      @pl.when(c0 + 2 < n_chunks)
      def _wait2():
        wait_edges(ebuf0, sem_e0)
      cnt2 = filt_chunk(ebuf0, csrc0, cdst0)
      drain_chunk(csrc1, cdst1, cnt1)

      @pl.when(c0 + 2 < n_chunks)
      def _fire2():
        fire_first(csrc0, cnt2)
      return cnt2

    lax.fori_loop(0, n_chunks // 2, pair_body, cnt0)

    pltpu.sync_copy(acc_v.at[pl.ds(0, n_per_tile)],
                    out_hbm.at[pl.ds(lo, n_per_tile)])

  return pl.kernel(
      body,
      out_type=jax.ShapeDtypeStruct((n_pad, d), jnp.float32),
      mesh=mesh,
      compiler_params=pltpu.CompilerParams(needs_layout_passes=False),
      scratch_types=[
          pltpu.VMEM((2, K_EDGES), jnp.int32),
          pltpu.VMEM((2, K_EDGES), jnp.int32),
          pltpu.VMEM((K_EDGES + B_ROWS,), jnp.int32),
          pltpu.VMEM((K_EDGES + B_ROWS,), jnp.int32),
          pltpu.VMEM((K_EDGES + B_ROWS,), jnp.int32),
          pltpu.VMEM((K_EDGES + B_ROWS,), jnp.int32),
          *[pltpu.VMEM((B_ROWS, d), jnp.float32) for _ in range(N_RING)],
          pltpu.SemaphoreType.DMA,
          pltpu.SemaphoreType.DMA,
          *[pltpu.SemaphoreType.DMA for _ in range(N_RING)],
          pltpu.VMEM((n_per_tile + 1, d), jnp.float32),
      ],
  )


def _layernorm(h, g, b):
  mu = jnp.mean(h, axis=-1, keepdims=True)
  var = jnp.mean((h - mu) ** 2, axis=-1, keepdims=True)
  return (h - mu) / jnp.sqrt(var + EPS) * g + b


def _tc1_body(agg_ref, x_ref, wl_ref, b_ref, wr_ref, g_ref, be_ref, o_ref):
  a = agg_ref[...]
  a = jnp.where(a == -jnp.inf, 0.0, a)
  h = (jnp.dot(a, wl_ref[...], preferred_element_type=jnp.float32)
       + jnp.dot(x_ref[...], wr_ref[...], preferred_element_type=jnp.float32)
       + b_ref[...])
  h = _layernorm(h, g_ref[...], be_ref[...])
  h = jnp.maximum(h, 0.0)
  o_ref[...] = jnp.concatenate([h, jnp.zeros_like(h)], axis=1)


def _tc2_body(agg_ref, h1_ref, wl_ref, b_ref, wr_ref, g_ref, be_ref,
              wm1_ref, bm1_ref, wm2_ref, bm2_ref, o_ref):
  d_h = wl_ref.shape[0]
  agg = agg_ref[...][:, :d_h]
  h1 = h1_ref[...][:, :d_h]
  h = (jnp.dot(agg, wl_ref[...], preferred_element_type=jnp.float32)
       + jnp.dot(h1, wr_ref[...], preferred_element_type=jnp.float32)
       + b_ref[...])
  h = _layernorm(h, g_ref[...], be_ref[...])
  h = jnp.maximum(h, 0.0)
  z = jnp.maximum(
      jnp.dot(h, wm1_ref[...], preferred_element_type=jnp.float32)
      + bm1_ref[...], 0.0)
  y = jnp.dot(z, wm2_ref[...], preferred_element_type=jnp.float32) + bm2_ref[...]
  o_ref[...] = jax.nn.sigmoid(y).reshape(o_ref.shape)


def _const_spec(shape):
  return pl.BlockSpec(shape, lambda i: (0,) * len(shape))


def kernel(x, edge_index, W_l1, b_l1, W_r1, W_l2, b_l2, W_r2,
           g1, be1, g2, be2, Wm1, bm1, Wm2, bm2):
  n, d_in = x.shape
  d_h = W_l1.shape[0]
  e = edge_index.shape[1]

  n_per_tile = ((n + NW - 1) // NW + 7) // 8 * 8
  n_pad = n_per_tile * NW
  e_pad = ((e + 2 * K_EDGES - 1) // (2 * K_EDGES)) * (2 * K_EDGES)

  src = edge_index[0].astype(jnp.int32)
  dst = edge_index[1].astype(jnp.int32)
  if e_pad != e:
    src = jnp.pad(src, (0, e_pad - e))
    dst = jnp.pad(dst, (0, e_pad - e), constant_values=n_pad)
  edges = jnp.stack([src, dst])
  x_pad = jnp.pad(x, ((0, n_pad - n), (0, 0)))

  segmax1 = _make_segmax(n_pad, n_per_tile, d_in, e_pad, -jnp.inf)
  segmax2 = _make_segmax(n_pad, n_per_tile, d_in, e_pad, 0.0)

  agg1 = segmax1(x_pad, edges)

  blk = 1024
  grid = (n_pad // blk,)
  row_spec = lambda dd: pl.BlockSpec((blk, dd), lambda i: (i, 0))

  h1 = pl.pallas_call(
      _tc1_body,
      grid=grid,
      in_specs=[row_spec(d_in), row_spec(d_in),
                _const_spec((d_in, d_h)), _const_spec((1, d_h)),
                _const_spec((d_in, d_h)), _const_spec((1, d_h)),
                _const_spec((1, d_h))],
      out_specs=row_spec(2 * d_h),
      out_shape=jax.ShapeDtypeStruct((n_pad, 2 * d_h), jnp.float32),
  )(agg1, x_pad, W_l1.T, b_l1.reshape(1, -1), W_r1.T,
    g1.reshape(1, -1), be1.reshape(1, -1))

  agg2 = segmax2(h1, edges)

  d_m = Wm1.shape[0]
  out = pl.pallas_call(
      _tc2_body,
      grid=grid,
      in_specs=[row_spec(2 * d_h), row_spec(2 * d_h),
                _const_spec((d_h, d_h)), _const_spec((1, d_h)),
                _const_spec((d_h, d_h)), _const_spec((1, d_h)),
                _const_spec((1, d_h)),
                _const_spec((d_h, d_m)), _const_spec((1, d_m)),
                _const_spec((d_m, 1)), _const_spec((1, 1))],
      out_specs=pl.BlockSpec((blk // 128, 128), lambda i: (i, 0)),
      out_shape=jax.ShapeDtypeStruct((n_pad // 128, 128), jnp.float32),
  )(agg2, h1, W_l2.T, b_l2.reshape(1, -1), W_r2.T,
    g2.reshape(1, -1), be2.reshape(1, -1),
    Wm1.T, bm1.reshape(1, -1), Wm2.T, bm2.reshape(1, -1))

  return out.reshape(-1)[:n]
